# baseline (speedup 1.0000x reference)
import jax
from jax.experimental import pallas as pl
from jax.experimental.pallas import tpu as pltpu


_ROWS = 320000
_BLOCK_ROWS = 4000


def _copy_kernel(x_ref, o_ref):
    o_ref[...] = x_ref[...]


def kernel(x_j):
    grid = (_ROWS // _BLOCK_ROWS,)
    return pl.pallas_call(
        _copy_kernel,
        out_shape=jax.ShapeDtypeStruct(x_j.shape, x_j.dtype),
        grid=grid,
        in_specs=[pl.BlockSpec((_BLOCK_ROWS, 128), lambda i: (i, 0))],
        out_specs=pl.BlockSpec((_BLOCK_ROWS, 128), lambda i: (i, 0)),
    )(x_j)


# speedup vs baseline: 1.1113x; 1.1113x over previous
import jax
from jax.experimental import pallas as pl
from jax.experimental.pallas import tpu as pltpu


_ROWS = 320000
_BLOCK_ROWS = 16000


def _copy_kernel(x_ref, o_ref):
    o_ref[...] = x_ref[...]


def kernel(x_j):
    grid = (_ROWS // _BLOCK_ROWS,)
    return pl.pallas_call(
        _copy_kernel,
        out_shape=jax.ShapeDtypeStruct(x_j.shape, x_j.dtype),
        grid=grid,
        in_specs=[pl.BlockSpec((_BLOCK_ROWS, 128), lambda i: (i, 0))],
        out_specs=pl.BlockSpec((_BLOCK_ROWS, 128), lambda i: (i, 0)),
    )(x_j)


# speedup vs baseline: 1.1166x; 1.0048x over previous
import jax
from jax.experimental import pallas as pl
from jax.experimental.pallas import tpu as pltpu


_ROWS = 320000
_BLOCK_ROWS = 20000


def _copy_kernel(x_ref, o_ref):
    o_ref[...] = x_ref[...]


def kernel(x_j):
    grid = (_ROWS // _BLOCK_ROWS,)
    return pl.pallas_call(
        _copy_kernel,
        out_shape=jax.ShapeDtypeStruct(x_j.shape, x_j.dtype),
        grid=grid,
        in_specs=[pl.BlockSpec((_BLOCK_ROWS, 128), lambda i: (i, 0))],
        out_specs=pl.BlockSpec((_BLOCK_ROWS, 128), lambda i: (i, 0)),
    )(x_j)
